# baseline (speedup 1.0000x reference)
import functools

import jax
import jax.numpy as jnp
from jax import lax
from jax.experimental import pallas as pl
from jax.experimental.pallas import tpu as pltpu
from jax.experimental.pallas import tpu_sc as plsc

LANES = 16
WIN = 512
NUM_TILES = 16
NUM_CORES = 2
ZROWS = 128


def _sc_accumulate(n_pad, edges_per_tile, x, srcs, dsts, rels):
    n_windows = edges_per_tile // WIN
    rows_out_per_tile = n_pad // NUM_TILES
    n_nodes = x.shape[0]
    n_dummy_groups = (n_pad - n_nodes) // LANES
    assert rows_out_per_tile % ZROWS == 0
    assert n_windows >= 2 and n_windows % 2 == 0

    mesh = plsc.VectorSubcoreMesh(core_axis_name="c", subcore_axis_name="s")

    @functools.partial(
        pl.kernel,
        mesh=mesh,
        compiler_params=pltpu.CompilerParams(use_tc_tiling_on_sc=False),
        out_type=jax.ShapeDtypeStruct((NUM_CORES * n_pad, LANES), jnp.float32),
        scratch_types=[
            [pltpu.VMEM((WIN,), jnp.int32) for _ in range(2)],
            [pltpu.VMEM((WIN,), jnp.int32) for _ in range(2)],
            [pltpu.VMEM((WIN,), jnp.int32) for _ in range(2)],
            [pltpu.VMEM((WIN, LANES), jnp.float32) for _ in range(2)],
            [pltpu.VMEM((WIN,), jnp.int32) for _ in range(2)],
            pltpu.VMEM((WIN,), jnp.float32),
            pltpu.VMEM((ZROWS, LANES), jnp.float32),
            pltpu.VMEM((ZROWS,), jnp.float32),
            pltpu.VMEM_SHARED((n_pad, LANES), jnp.float32),
            pltpu.VMEM_SHARED((n_pad,), jnp.float32),
            [pltpu.SemaphoreType.DMA for _ in range(2)],
            [pltpu.SemaphoreType.DMA for _ in range(2)],
            [pltpu.SemaphoreType.DMA for _ in range(2)],
        ],
    )
    def sc_kernel(x_hbm, src_hbm, dst_hbm, rel_hbm, acc_out,
                  src_v, dst_v, rel_v, rows_v, idx_v, ones_v, zrow_v, zdeg_v,
                  acc_sh, deg_sh, semg, seml, sems):
        c = lax.axis_index("c")
        s = lax.axis_index("s")

        ones16 = jnp.ones((LANES,), jnp.float32)
        zero16 = jnp.zeros((LANES,), jnp.float32)

        def fill_ones_body(i, _):
            ones_v[pl.ds(i * LANES, LANES)] = ones16
            return 0
        lax.fori_loop(0, WIN // LANES, fill_ones_body, 0)

        def fill_zrow(i, _):
            zrow_v[i] = zero16
            return 0
        lax.fori_loop(0, ZROWS, fill_zrow, 0)

        def fill_zdeg(i, _):
            zdeg_v[pl.ds(i * LANES, LANES)] = zero16
            return 0
        lax.fori_loop(0, ZROWS // LANES, fill_zdeg, 0)

        out_base = s * rows_out_per_tile
        for k in range(rows_out_per_tile // ZROWS):
            pltpu.sync_copy(zrow_v, acc_sh.at[pl.ds(out_base + k * ZROWS, ZROWS)])
            pltpu.sync_copy(zdeg_v, deg_sh.at[pl.ds(out_base + k * ZROWS, ZROWS)])
        plsc.subcore_barrier()

        iota16 = lax.iota(jnp.int32, LANES)
        tile_edge0 = s * edges_per_tile
        last_w = n_windows - 1

        def edge_base(w):
            wc = jnp.minimum(w, last_w)
            return tile_edge0 + wc * WIN

        def issue_loads(w, q):
            b = edge_base(w)
            pltpu.async_copy(src_hbm.at[pl.ds(b, WIN)], src_v[q], seml[q])
            pltpu.async_copy(dst_hbm.at[pl.ds(b, WIN)], dst_v[q], seml[q])
            pltpu.async_copy(rel_hbm.at[pl.ds(b, WIN)], rel_v[q], seml[q])

        def wait_loads(q):
            pltpu.make_async_copy(src_hbm.at[pl.ds(0, WIN)], src_v[q], seml[q]).wait()
            pltpu.make_async_copy(dst_hbm.at[pl.ds(0, WIN)], dst_v[q], seml[q]).wait()
            pltpu.make_async_copy(rel_hbm.at[pl.ds(0, WIN)], rel_v[q], seml[q]).wait()

        def issue_gather(q):
            pltpu.async_copy(x_hbm.at[src_v[q]], rows_v[q], semg[q])

        def wait_gather(q):
            pltpu.make_async_copy(x_hbm.at[src_v[q]], rows_v[q], semg[q]).wait()

        def issue_scatter(q):
            pltpu.async_copy(rows_v[q], acc_sh.at[idx_v[q]], sems[q], add=True)
            pltpu.async_copy(ones_v, deg_sh.at[idx_v[q]], sems[q], add=True)

        def wait_scatter(q):
            pltpu.make_async_copy(rows_v[q], acc_sh.at[idx_v[q]], sems[q]).wait()
            pltpu.make_async_copy(ones_v, deg_sh.at[idx_v[q]], sems[q]).wait()

        issue_loads(0, 0)
        wait_loads(0)
        issue_gather(0)
        issue_loads(1, 1)

        def outer(k, _):
            for p in (0, 1):
                w = 2 * k + p
                if p == 0:
                    @pl.when(k > 0)
                    def _():
                        wait_scatter(0)
                else:
                    @pl.when(k > 0)
                    def _():
                        wait_scatter(1)
                wait_gather(p)
                dummy16 = n_nodes + ((w + s) % n_dummy_groups) * LANES + iota16
                for t in range(WIN // LANES):
                    rl = rel_v[p][pl.ds(t * LANES, LANES)]
                    dv = dst_v[p][pl.ds(t * LANES, LANES)]
                    idx_v[p][pl.ds(t * LANES, LANES)] = jnp.where(rl == c, dv, dummy16)
                wait_loads(1 - p)
                issue_gather(1 - p)
                issue_loads(w + 2, p)
                issue_scatter(p)
            return 0

        lax.fori_loop(0, n_windows // 2, outer, 0)
        wait_scatter(0)
        wait_scatter(1)
        wait_gather(0)
        wait_loads(1)
        plsc.subcore_barrier()

        def norm_chunk(k, _):
            off = out_base + k * ZROWS
            pltpu.sync_copy(acc_sh.at[pl.ds(off, ZROWS)], zrow_v)
            pltpu.sync_copy(deg_sh.at[pl.ds(off, ZROWS)], ones_v.at[pl.ds(0, ZROWS)])
            for g in range(ZROWS // LANES):
                dv = ones_v[pl.ds(g * LANES, LANES)]
                inv = 1.0 / jnp.maximum(dv, 1.0)
                for i in range(LANES):
                    r = g * LANES + i
                    zrow_v[r] = zrow_v[r] * jnp.full((LANES,), inv[i])
            pltpu.sync_copy(zrow_v, acc_out.at[pl.ds(c * n_pad + off, ZROWS)])
            return 0
        lax.fori_loop(0, rows_out_per_tile // ZROWS, norm_chunk, 0)

    hflat = sc_kernel(x, srcs, dsts, rels)
    return hflat


def _tc_finish(hflat, weights, n_nodes, n_pad, block):
    assert n_pad % block == 0
    n_blocks_half = n_pad // block
    grid = -(-n_nodes // block)

    def body(a0_ref, a1_ref, w_ref, o_ref):
        w = w_ref[...]
        y = (jnp.dot(a0_ref[...], w[0], preferred_element_type=jnp.float32)
             + jnp.dot(a1_ref[...], w[1], preferred_element_type=jnp.float32))
        o_ref[...] = jnp.maximum(y, 0.0)

    return pl.pallas_call(
        body,
        grid=(grid,),
        in_specs=[
            pl.BlockSpec((block, LANES), lambda i: (i, 0)),
            pl.BlockSpec((block, LANES), lambda i: (i + n_blocks_half, 0)),
            pl.BlockSpec((NUM_CORES, LANES, LANES), lambda i: (0, 0, 0)),
        ],
        out_specs=pl.BlockSpec((block, LANES), lambda i: (i, 0)),
        out_shape=jax.ShapeDtypeStruct((n_nodes, LANES), jnp.float32),
    )(hflat, hflat, weights)


@jax.jit
def kernel(x, edge_index, rel, weights):
    n_nodes = x.shape[0]
    n_edges = edge_index.shape[1]
    n_rel = weights.shape[0]
    assert n_rel == NUM_CORES

    src = edge_index[0].astype(jnp.int32)
    dst = edge_index[1].astype(jnp.int32)
    rel32 = rel.astype(jnp.int32)

    per_tile = -(-n_edges // (NUM_TILES * 2 * WIN)) * 2 * WIN
    e_pad = per_tile * NUM_TILES
    pad = e_pad - n_edges
    pad_src = jnp.arange(pad, dtype=jnp.int32) % n_nodes
    src_p = jnp.concatenate([src, pad_src])
    dst_p = jnp.concatenate([dst, jnp.zeros((pad,), jnp.int32)])
    rel_p = jnp.concatenate([rel32, jnp.full((pad,), n_rel, jnp.int32)])

    n_pad = -(-(n_nodes + LANES) // (NUM_TILES * ZROWS)) * (NUM_TILES * ZROWS)

    hflat = _sc_accumulate(n_pad, per_tile, x, src_p, dst_p, rel_p)
    return _tc_finish(hflat, weights, n_nodes, n_pad, block=6272)


# speedup vs baseline: 1.1426x; 1.1426x over previous
import functools

import jax
import jax.numpy as jnp
from jax import lax
from jax.experimental import pallas as pl
from jax.experimental.pallas import tpu as pltpu
from jax.experimental.pallas import tpu_sc as plsc

LANES = 16
WIN = 512
NUM_TILES = 16
NUM_CORES = 2
ZROWS = 128


def _sc_accumulate(n_pad, edges_per_tile, x, srcs, dsts, rels):
    n_windows = edges_per_tile // WIN
    rows_out_per_tile = n_pad // NUM_TILES
    n_nodes = x.shape[0]
    n_dummy_groups = (n_pad - n_nodes) // LANES
    assert rows_out_per_tile % ZROWS == 0
    assert n_windows >= 2 and n_windows % 2 == 0

    mesh = plsc.VectorSubcoreMesh(core_axis_name="c", subcore_axis_name="s")

    @functools.partial(
        pl.kernel,
        mesh=mesh,
        compiler_params=pltpu.CompilerParams(use_tc_tiling_on_sc=False),
        out_type=jax.ShapeDtypeStruct((NUM_CORES * n_pad, LANES), jnp.float32),
        scratch_types=[
            [pltpu.VMEM((WIN,), jnp.int32) for _ in range(2)],
            [pltpu.VMEM((WIN,), jnp.int32) for _ in range(2)],
            [pltpu.VMEM((WIN,), jnp.int32) for _ in range(2)],
            [pltpu.VMEM((WIN, LANES), jnp.float32) for _ in range(2)],
            [pltpu.VMEM((WIN,), jnp.int32) for _ in range(2)],
            pltpu.VMEM((WIN,), jnp.float32),
            pltpu.VMEM((ZROWS, LANES), jnp.float32),
            pltpu.VMEM((ZROWS,), jnp.float32),
            pltpu.VMEM_SHARED((n_pad, LANES), jnp.float32),
            pltpu.VMEM_SHARED((n_pad,), jnp.float32),
            [pltpu.SemaphoreType.DMA for _ in range(2)],
            [pltpu.SemaphoreType.DMA for _ in range(2)],
            [pltpu.SemaphoreType.DMA for _ in range(2)],
        ],
    )
    def sc_kernel(x_hbm, src_hbm, dst_hbm, rel_hbm, acc_out,
                  src_v, dst_v, rel_v, rows_v, idx_v, ones_v, zrow_v, zdeg_v,
                  acc_sh, deg_sh, semg, seml, sems):
        c = lax.axis_index("c")
        s = lax.axis_index("s")

        ones16 = jnp.ones((LANES,), jnp.float32)
        zero16 = jnp.zeros((LANES,), jnp.float32)

        def fill_ones_body(i, _):
            ones_v[pl.ds(i * LANES, LANES)] = ones16
            return 0
        lax.fori_loop(0, WIN // LANES, fill_ones_body, 0)

        def fill_zrow(i, _):
            zrow_v[i] = zero16
            return 0
        lax.fori_loop(0, ZROWS, fill_zrow, 0)

        def fill_zdeg(i, _):
            zdeg_v[pl.ds(i * LANES, LANES)] = zero16
            return 0
        lax.fori_loop(0, ZROWS // LANES, fill_zdeg, 0)

        out_base = s * rows_out_per_tile
        for k in range(rows_out_per_tile // ZROWS):
            pltpu.sync_copy(zrow_v, acc_sh.at[pl.ds(out_base + k * ZROWS, ZROWS)])
            pltpu.sync_copy(zdeg_v, deg_sh.at[pl.ds(out_base + k * ZROWS, ZROWS)])
        plsc.subcore_barrier()

        iota16 = lax.iota(jnp.int32, LANES)
        tile_edge0 = s * edges_per_tile
        last_w = n_windows - 1

        def edge_base(w):
            wc = jnp.minimum(w, last_w)
            return tile_edge0 + wc * WIN

        def issue_loads(w, q):
            b = edge_base(w)
            pltpu.async_copy(src_hbm.at[pl.ds(b, WIN)], src_v[q], seml[q])
            pltpu.async_copy(dst_hbm.at[pl.ds(b, WIN)], dst_v[q], seml[q])
            pltpu.async_copy(rel_hbm.at[pl.ds(b, WIN)], rel_v[q], seml[q])

        def wait_loads(q):
            pltpu.make_async_copy(src_hbm.at[pl.ds(0, WIN)], src_v[q], seml[q]).wait()
            pltpu.make_async_copy(dst_hbm.at[pl.ds(0, WIN)], dst_v[q], seml[q]).wait()
            pltpu.make_async_copy(rel_hbm.at[pl.ds(0, WIN)], rel_v[q], seml[q]).wait()

        def issue_gather(q):
            pltpu.async_copy(x_hbm.at[src_v[q]], rows_v[q], semg[q])

        def wait_gather(q):
            pltpu.make_async_copy(x_hbm.at[src_v[q]], rows_v[q], semg[q]).wait()

        def issue_scatter(q):
            pltpu.async_copy(rows_v[q], acc_sh.at[idx_v[q]], sems[q], add=True)
            pltpu.async_copy(ones_v, deg_sh.at[idx_v[q]], sems[q], add=True)

        def wait_scatter(q):
            pltpu.make_async_copy(rows_v[q], acc_sh.at[idx_v[q]], sems[q]).wait()
            pltpu.make_async_copy(ones_v, deg_sh.at[idx_v[q]], sems[q]).wait()

        issue_loads(0, 0)
        wait_loads(0)
        issue_gather(0)
        issue_loads(1, 1)

        def outer(k, _):
            for p in (0, 1):
                w = 2 * k + p
                if p == 0:
                    @pl.when(k > 0)
                    def _():
                        wait_scatter(0)
                else:
                    @pl.when(k > 0)
                    def _():
                        wait_scatter(1)
                wait_gather(p)
                dummy16 = n_nodes + ((w + s) % n_dummy_groups) * LANES + iota16
                for t in range(WIN // LANES):
                    rl = rel_v[p][pl.ds(t * LANES, LANES)]
                    dv = dst_v[p][pl.ds(t * LANES, LANES)]
                    idx_v[p][pl.ds(t * LANES, LANES)] = jnp.where(rl == c, dv, dummy16)
                wait_loads(1 - p)
                issue_gather(1 - p)
                issue_loads(w + 2, p)
                issue_scatter(p)
            return 0

        lax.fori_loop(0, n_windows // 2, outer, 0)
        wait_scatter(0)
        wait_scatter(1)
        wait_gather(0)
        wait_loads(1)
        plsc.subcore_barrier()

        def norm_chunk(k, _):
            off = out_base + k * ZROWS
            pltpu.sync_copy(acc_sh.at[pl.ds(off, ZROWS)], zrow_v)
            pltpu.sync_copy(deg_sh.at[pl.ds(off, ZROWS)], ones_v.at[pl.ds(0, ZROWS)])
            for g in range(ZROWS // LANES):
                dv = ones_v[pl.ds(g * LANES, LANES)]
                inv = 1.0 / jnp.maximum(dv, 1.0)
                for i in range(LANES):
                    r = g * LANES + i
                    zrow_v[r] = zrow_v[r] * jnp.full((LANES,), inv[i])
            pltpu.sync_copy(zrow_v, acc_out.at[pl.ds(c * n_pad + off, ZROWS)])
            return 0
        lax.fori_loop(0, rows_out_per_tile // ZROWS, norm_chunk, 0)

    hflat = sc_kernel(x, srcs, dsts, rels)
    return hflat


def _tc_finish(hflat, weights, n_nodes, n_pad, block):
    assert n_pad % block == 0 and block % 8 == 0
    n_blocks_half = n_pad // block
    grid = n_pad // block
    hp = hflat.reshape(NUM_CORES * n_pad // 8, 8 * LANES)
    eye8 = jnp.eye(8, dtype=jnp.float32)
    wk = jax.vmap(lambda w: jnp.kron(eye8, w))(weights)

    def body(a0_ref, a1_ref, w_ref, o_ref):
        w = w_ref[...]
        y = (jnp.dot(a0_ref[...], w[0], preferred_element_type=jnp.float32)
             + jnp.dot(a1_ref[...], w[1], preferred_element_type=jnp.float32))
        o_ref[...] = jnp.maximum(y, 0.0)

    yp = pl.pallas_call(
        body,
        grid=(grid,),
        in_specs=[
            pl.BlockSpec((block // 8, 8 * LANES), lambda i: (i, 0)),
            pl.BlockSpec((block // 8, 8 * LANES), lambda i: (i + n_blocks_half, 0)),
            pl.BlockSpec((NUM_CORES, 8 * LANES, 8 * LANES), lambda i: (0, 0, 0)),
        ],
        out_specs=pl.BlockSpec((block // 8, 8 * LANES), lambda i: (i, 0)),
        out_shape=jax.ShapeDtypeStruct((n_pad // 8, 8 * LANES), jnp.float32),
    )(hp, hp, wk)
    return yp.reshape(n_pad, LANES)[:n_nodes]


@jax.jit
def kernel(x, edge_index, rel, weights):
    n_nodes = x.shape[0]
    n_edges = edge_index.shape[1]
    n_rel = weights.shape[0]
    assert n_rel == NUM_CORES

    src = edge_index[0].astype(jnp.int32)
    dst = edge_index[1].astype(jnp.int32)
    rel32 = rel.astype(jnp.int32)

    per_tile = -(-n_edges // (NUM_TILES * 2 * WIN)) * 2 * WIN
    e_pad = per_tile * NUM_TILES
    pad = e_pad - n_edges
    pad_src = jnp.arange(pad, dtype=jnp.int32) % n_nodes
    src_p = jnp.concatenate([src, pad_src])
    dst_p = jnp.concatenate([dst, jnp.zeros((pad,), jnp.int32)])
    rel_p = jnp.concatenate([rel32, jnp.full((pad,), n_rel, jnp.int32)])

    n_pad = -(-(n_nodes + LANES) // (NUM_TILES * ZROWS)) * (NUM_TILES * ZROWS)

    hflat = _sc_accumulate(n_pad, per_tile, x, src_p, dst_p, rel_p)
    return _tc_finish(hflat, weights, n_nodes, n_pad, block=6272)


# speedup vs baseline: 1.1603x; 1.0154x over previous
import functools

import jax
import jax.numpy as jnp
from jax import lax
from jax.experimental import pallas as pl
from jax.experimental.pallas import tpu as pltpu
from jax.experimental.pallas import tpu_sc as plsc

LANES = 16
WIN = 512
NUM_TILES = 16
NUM_CORES = 2
ZROWS = 128


def _sc_accumulate(n_pad, edges_per_tile, x, srcs, dsts, rels):
    n_windows = edges_per_tile // WIN
    rows_out_per_tile = n_pad // NUM_TILES
    n_nodes = x.shape[0]
    n_dummy_groups = (n_pad - n_nodes) // LANES
    assert rows_out_per_tile % ZROWS == 0
    assert n_windows >= 2 and n_windows % 2 == 0

    mesh = plsc.VectorSubcoreMesh(core_axis_name="c", subcore_axis_name="s")

    @functools.partial(
        pl.kernel,
        mesh=mesh,
        compiler_params=pltpu.CompilerParams(use_tc_tiling_on_sc=False),
        out_type=jax.ShapeDtypeStruct((NUM_CORES * n_pad, LANES), jnp.float32),
        scratch_types=[
            [pltpu.VMEM((WIN,), jnp.int32) for _ in range(2)],
            [pltpu.VMEM((WIN,), jnp.int32) for _ in range(2)],
            [pltpu.VMEM((WIN,), jnp.int32) for _ in range(2)],
            [pltpu.VMEM((WIN, LANES), jnp.float32) for _ in range(2)],
            [pltpu.VMEM((WIN,), jnp.int32) for _ in range(2)],
            pltpu.VMEM((WIN,), jnp.float32),
            pltpu.VMEM((ZROWS, LANES), jnp.float32),
            pltpu.VMEM((ZROWS,), jnp.float32),
            pltpu.VMEM_SHARED((n_pad, LANES), jnp.float32),
            pltpu.VMEM_SHARED((n_pad,), jnp.float32),
            [pltpu.SemaphoreType.DMA for _ in range(2)],
            [pltpu.SemaphoreType.DMA for _ in range(2)],
            [pltpu.SemaphoreType.DMA for _ in range(2)],
        ],
    )
    def sc_kernel(x_hbm, src_hbm, dst_hbm, rel_hbm, acc_out,
                  src_v, dst_v, rel_v, rows_v, idx_v, ones_v, zrow_v, zdeg_v,
                  acc_sh, deg_sh, semg, seml, sems):
        c = lax.axis_index("c")
        s = lax.axis_index("s")

        ones16 = jnp.ones((LANES,), jnp.float32)
        zero16 = jnp.zeros((LANES,), jnp.float32)

        def fill_ones_body(i, _):
            ones_v[pl.ds(i * LANES, LANES)] = ones16
            return 0
        lax.fori_loop(0, WIN // LANES, fill_ones_body, 0)

        def fill_zrow(i, _):
            zrow_v[i] = zero16
            return 0
        lax.fori_loop(0, ZROWS, fill_zrow, 0)

        def fill_zdeg(i, _):
            zdeg_v[pl.ds(i * LANES, LANES)] = zero16
            return 0
        lax.fori_loop(0, ZROWS // LANES, fill_zdeg, 0)

        out_base = s * rows_out_per_tile
        for k in range(rows_out_per_tile // ZROWS):
            pltpu.sync_copy(zrow_v, acc_sh.at[pl.ds(out_base + k * ZROWS, ZROWS)])
            pltpu.sync_copy(zdeg_v, deg_sh.at[pl.ds(out_base + k * ZROWS, ZROWS)])
        plsc.subcore_barrier()

        iota16 = lax.iota(jnp.int32, LANES)
        tile_edge0 = s * edges_per_tile
        last_w = n_windows - 1

        def edge_base(w):
            wc = jnp.minimum(w, last_w)
            return tile_edge0 + wc * WIN

        def issue_loads(w, q):
            b = edge_base(w)
            pltpu.async_copy(src_hbm.at[pl.ds(b, WIN)], src_v[q], seml[q])
            pltpu.async_copy(dst_hbm.at[pl.ds(b, WIN)], dst_v[q], seml[q])
            pltpu.async_copy(rel_hbm.at[pl.ds(b, WIN)], rel_v[q], seml[q])

        def wait_loads(q):
            pltpu.make_async_copy(src_hbm.at[pl.ds(0, WIN)], src_v[q], seml[q]).wait()
            pltpu.make_async_copy(dst_hbm.at[pl.ds(0, WIN)], dst_v[q], seml[q]).wait()
            pltpu.make_async_copy(rel_hbm.at[pl.ds(0, WIN)], rel_v[q], seml[q]).wait()

        def issue_gather(q):
            pltpu.async_copy(x_hbm.at[src_v[q]], rows_v[q], semg[q])

        def wait_gather(q):
            pltpu.make_async_copy(x_hbm.at[src_v[q]], rows_v[q], semg[q]).wait()

        def issue_scatter(q):
            pltpu.async_copy(rows_v[q], acc_sh.at[idx_v[q]], sems[q], add=True)
            pltpu.async_copy(ones_v, deg_sh.at[idx_v[q]], sems[q], add=True)

        def wait_scatter(q):
            pltpu.make_async_copy(rows_v[q], acc_sh.at[idx_v[q]], sems[q]).wait()
            pltpu.make_async_copy(ones_v, deg_sh.at[idx_v[q]], sems[q]).wait()

        issue_loads(0, 0)
        wait_loads(0)
        issue_gather(0)
        issue_loads(1, 1)

        def outer(k, _):
            for p in (0, 1):
                w = 2 * k + p
                if p == 0:
                    @pl.when(k > 0)
                    def _():
                        wait_scatter(0)
                else:
                    @pl.when(k > 0)
                    def _():
                        wait_scatter(1)
                wait_gather(p)
                wait_loads(1 - p)
                issue_gather(1 - p)
                dummy16 = n_nodes + ((w + s) % n_dummy_groups) * LANES + iota16
                for t in range(WIN // LANES):
                    rl = rel_v[p][pl.ds(t * LANES, LANES)]
                    dv = dst_v[p][pl.ds(t * LANES, LANES)]
                    idx_v[p][pl.ds(t * LANES, LANES)] = jnp.where(rl == c, dv, dummy16)
                issue_loads(w + 2, p)
                issue_scatter(p)
            return 0

        lax.fori_loop(0, n_windows // 2, outer, 0)
        wait_scatter(0)
        wait_scatter(1)
        wait_gather(0)
        wait_loads(1)
        plsc.subcore_barrier()

        def norm_chunk(k, _):
            off = out_base + k * ZROWS
            pltpu.sync_copy(acc_sh.at[pl.ds(off, ZROWS)], zrow_v)
            pltpu.sync_copy(deg_sh.at[pl.ds(off, ZROWS)], ones_v.at[pl.ds(0, ZROWS)])
            for g in range(ZROWS // LANES):
                dv = ones_v[pl.ds(g * LANES, LANES)]
                inv = 1.0 / jnp.maximum(dv, 1.0)
                for i in range(LANES):
                    r = g * LANES + i
                    zrow_v[r] = zrow_v[r] * jnp.full((LANES,), inv[i])
            pltpu.sync_copy(zrow_v, acc_out.at[pl.ds(c * n_pad + off, ZROWS)])
            return 0
        lax.fori_loop(0, rows_out_per_tile // ZROWS, norm_chunk, 0)

    hflat = sc_kernel(x, srcs, dsts, rels)
    return hflat


def _tc_finish(hflat, weights, n_nodes, n_pad, block):
    assert n_pad % block == 0 and block % 8 == 0
    n_blocks_half = n_pad // block
    grid = n_pad // block
    hp = hflat.reshape(NUM_CORES * n_pad // 8, 8 * LANES)
    eye8 = jnp.eye(8, dtype=jnp.float32)
    wk = jax.vmap(lambda w: jnp.kron(eye8, w))(weights)

    def body(a0_ref, a1_ref, w_ref, o_ref):
        w = w_ref[...]
        y = (jnp.dot(a0_ref[...], w[0], preferred_element_type=jnp.float32)
             + jnp.dot(a1_ref[...], w[1], preferred_element_type=jnp.float32))
        o_ref[...] = jnp.maximum(y, 0.0)

    yp = pl.pallas_call(
        body,
        grid=(grid,),
        in_specs=[
            pl.BlockSpec((block // 8, 8 * LANES), lambda i: (i, 0)),
            pl.BlockSpec((block // 8, 8 * LANES), lambda i: (i + n_blocks_half, 0)),
            pl.BlockSpec((NUM_CORES, 8 * LANES, 8 * LANES), lambda i: (0, 0, 0)),
        ],
        out_specs=pl.BlockSpec((block // 8, 8 * LANES), lambda i: (i, 0)),
        out_shape=jax.ShapeDtypeStruct((n_pad // 8, 8 * LANES), jnp.float32),
    )(hp, hp, wk)
    return yp.reshape(n_pad, LANES)[:n_nodes]


@jax.jit
def kernel(x, edge_index, rel, weights):
    n_nodes = x.shape[0]
    n_edges = edge_index.shape[1]
    n_rel = weights.shape[0]
    assert n_rel == NUM_CORES

    src = edge_index[0].astype(jnp.int32)
    dst = edge_index[1].astype(jnp.int32)
    rel32 = rel.astype(jnp.int32)

    per_tile = -(-n_edges // (NUM_TILES * 2 * WIN)) * 2 * WIN
    e_pad = per_tile * NUM_TILES
    pad = e_pad - n_edges
    pad_src = jnp.arange(pad, dtype=jnp.int32) % n_nodes
    src_p = jnp.concatenate([src, pad_src])
    dst_p = jnp.concatenate([dst, jnp.zeros((pad,), jnp.int32)])
    rel_p = jnp.concatenate([rel32, jnp.full((pad,), n_rel, jnp.int32)])

    n_pad = -(-(n_nodes + LANES) // (NUM_TILES * ZROWS)) * (NUM_TILES * ZROWS)

    hflat = _sc_accumulate(n_pad, per_tile, x, src_p, dst_p, rel_p)
    return _tc_finish(hflat, weights, n_nodes, n_pad, block=6272)
